# Initial kernel scaffold; baseline (speedup 1.0000x reference)
import functools

import jax
import jax.numpy as jnp
from jax import lax
from jax.experimental import pallas as pl
from jax.experimental.pallas import tpu as pltpu
from jax.experimental.pallas import tpu_sc as plsc

NUM_TABLE_ROWS = 1_000_000
DIM = 64
NC = 2
NS = 16
NW = NC * NS

K = 8
CHUNK = 128
GROUP = K * CHUNK


@functools.partial(jax.jit, static_argnames=("n_groups",))
def _emb_lookup(idx, table, n_groups):
    mesh = plsc.VectorSubcoreMesh(core_axis_name="c", subcore_axis_name="s")

    @functools.partial(
        pl.kernel,
        mesh=mesh,
        out_type=jax.ShapeDtypeStruct((NW, n_groups, K, CHUNK, DIM), jnp.float32),
        scratch_types=[
            pltpu.VMEM((K, CHUNK), jnp.int32),
            pltpu.VMEM((K, CHUNK, DIM), jnp.float32),
            pltpu.SemaphoreType.DMA,
        ],
    )
    def body(idx_hbm, table_hbm, out_hbm, idx_v, rows_v, sem):
        wid = lax.axis_index("s") * NC + lax.axis_index("c")

        def group(g, carry):
            pltpu.sync_copy(idx_hbm.at[wid, g], idx_v)
            descs = [
                pltpu.async_copy(table_hbm.at[idx_v.at[j]], rows_v.at[j], sem)
                for j in range(K)
            ]
            for d in descs:
                d.wait()
            pltpu.sync_copy(rows_v, out_hbm.at[wid, g])
            return carry

        lax.fori_loop(0, n_groups, group, 0)

    return body(idx, table)


def kernel(token_ids, embeddings):
    b, t = token_ids.shape
    total = b * t
    assert total % (NW * GROUP) == 0
    n_groups = total // (NW * GROUP)
    idx = token_ids.astype(jnp.int32).reshape(NW, n_groups, K, CHUNK)
    out = _emb_lookup(idx, embeddings, n_groups)
    return out.reshape(b, t, DIM)


# speedup vs baseline: 1.8467x; 1.8467x over previous
import functools

import jax
import jax.numpy as jnp
from jax import lax
from jax.experimental import pallas as pl
from jax.experimental.pallas import tpu as pltpu
from jax.experimental.pallas import tpu_sc as plsc

NUM_TABLE_ROWS = 1_000_000
DIM = 64
NC = 2
NS = 16
NW = NC * NS

K = 8
CHUNK = 128
GROUP = K * CHUNK


@functools.partial(jax.jit, static_argnames=("n_groups",))
def _emb_lookup(idx, table, n_groups):
    mesh = plsc.VectorSubcoreMesh(core_axis_name="c", subcore_axis_name="s")

    @functools.partial(
        pl.kernel,
        mesh=mesh,
        out_type=jax.ShapeDtypeStruct((NW, n_groups, K, CHUNK, DIM), jnp.float32),
        scratch_types=[
            pltpu.VMEM((K, CHUNK), jnp.int32),
            pltpu.VMEM((K, CHUNK, DIM), jnp.float32),
            pltpu.SemaphoreType.DMA,
        ],
        compiler_params=pltpu.CompilerParams(use_tc_tiling_on_sc=False),
    )
    def body(idx_hbm, table_hbm, out_hbm, idx_v, rows_v, sem):
        wid = lax.axis_index("s") * NC + lax.axis_index("c")

        def group(g, carry):
            pltpu.sync_copy(idx_hbm.at[wid, g], idx_v)
            descs = [
                pltpu.async_copy(table_hbm.at[idx_v.at[j]], rows_v.at[j], sem)
                for j in range(K)
            ]
            for d in descs:
                d.wait()
            pltpu.sync_copy(rows_v, out_hbm.at[wid, g])
            return carry

        lax.fori_loop(0, n_groups, group, 0)

    return body(idx, table)


def kernel(token_ids, embeddings):
    b, t = token_ids.shape
    total = b * t
    assert total % (NW * GROUP) == 0
    n_groups = total // (NW * GROUP)
    idx = token_ids.astype(jnp.int32).reshape(NW, n_groups, K, CHUNK)
    out = _emb_lookup(idx, embeddings, n_groups)
    return out.reshape(b, t, DIM)


# speedup vs baseline: 1.8732x; 1.0144x over previous
import functools

import jax
import jax.numpy as jnp
from jax import lax
from jax.experimental import pallas as pl
from jax.experimental.pallas import tpu as pltpu
from jax.experimental.pallas import tpu_sc as plsc

DIM = 64
NC = 2
NS = 16
NW = NC * NS

CHUNK = 128
K = 5
GROUP = K * CHUNK


@functools.partial(jax.jit, static_argnames=("n_groups",))
def _emb_lookup(idx, table, n_groups):
    mesh = plsc.VectorSubcoreMesh(core_axis_name="c", subcore_axis_name="s")
    n_chunks = n_groups * K

    @functools.partial(
        pl.kernel,
        mesh=mesh,
        out_type=jax.ShapeDtypeStruct((NW, n_groups, K, CHUNK, DIM), jnp.float32),
        scratch_types=[
            pltpu.VMEM((n_chunks, CHUNK), jnp.int32),
            pltpu.VMEM((2, K, CHUNK, DIM), jnp.float32),
            pltpu.SemaphoreType.DMA,
            pltpu.SemaphoreType.DMA,
        ],
        compiler_params=pltpu.CompilerParams(use_tc_tiling_on_sc=False),
    )
    def body(idx_hbm, table_hbm, out_hbm, idx_v, rows_v, sem0, sem1):
        wid = lax.axis_index("s") * NC + lax.axis_index("c")
        sems = (sem0, sem1)

        pltpu.sync_copy(idx_hbm.at[wid], idx_v)

        def fire(g, buf):
            sem = sems[buf]
            return [
                pltpu.async_copy(
                    table_hbm.at[idx_v.at[g * K + j]],
                    rows_v.at[buf, j],
                    sem,
                )
                for j in range(K)
            ]

        def drain_and_store(g, buf):
            sem = sems[buf]
            for j in range(K):
                pltpu.make_async_copy(
                    table_hbm.at[idx_v.at[j]], rows_v.at[buf, j], sem
                ).wait()
            pltpu.sync_copy(rows_v.at[buf], out_hbm.at[wid, g])

        fire(0, 0)
        n_outer = n_groups // 2

        def outer(t, carry):
            g0 = 2 * t
            fire(g0 + 1, 1)
            drain_and_store(g0, 0)

            @pl.when(t + 1 < n_outer)
            def _():
                fire(g0 + 2, 0)

            drain_and_store(g0 + 1, 1)
            return carry

        lax.fori_loop(0, n_outer, outer, 0)

    return body(idx, table)


def kernel(token_ids, embeddings):
    b, t = token_ids.shape
    total = b * t
    assert total % (NW * GROUP * 2) == 0
    n_groups = total // (NW * GROUP)
    idx = token_ids.astype(jnp.int32).reshape(NW, n_groups * K, CHUNK)
    out = _emb_lookup(idx, embeddings, n_groups)
    return out.reshape(b, t, DIM)


# speedup vs baseline: 1.8803x; 1.0038x over previous
import functools

import jax
import jax.numpy as jnp
from jax import lax
from jax.experimental import pallas as pl
from jax.experimental.pallas import tpu as pltpu
from jax.experimental.pallas import tpu_sc as plsc

DIM = 64
NC = 2
NS = 16
NW = NC * NS

NBG = 16


def _emb_lookup(idx, table):
    mesh = plsc.VectorSubcoreMesh(core_axis_name="c", subcore_axis_name="s")
    nbatch, seq = idx.shape
    b_per_w = nbatch // NW
    n_groups = b_per_w // NBG

    @functools.partial(
        pl.kernel,
        mesh=mesh,
        out_type=jax.ShapeDtypeStruct((nbatch, seq, DIM), jnp.float32),
        scratch_types=[
            pltpu.VMEM((b_per_w, seq), jnp.int32),
            pltpu.VMEM((2, NBG, seq, DIM), jnp.float32),
            pltpu.SemaphoreType.DMA,
            pltpu.SemaphoreType.DMA,
        ],
        compiler_params=pltpu.CompilerParams(use_tc_tiling_on_sc=False),
    )
    def body(idx_hbm, table_hbm, out_hbm, idx_v, rows_v, sem0, sem1):
        wid = lax.axis_index("s") * NC + lax.axis_index("c")
        base_b = wid * b_per_w
        sems = (sem0, sem1)

        pltpu.sync_copy(idx_hbm.at[pl.ds(base_b, b_per_w)], idx_v)

        def fire(g, buf):
            sem = sems[buf]
            for i in range(NBG):
                pltpu.async_copy(
                    table_hbm.at[idx_v.at[g * NBG + i]],
                    rows_v.at[buf, i],
                    sem,
                )

        def drain_and_store(g, buf):
            sem = sems[buf]
            for i in range(NBG):
                pltpu.make_async_copy(
                    table_hbm.at[idx_v.at[i]], rows_v.at[buf, i], sem
                ).wait()
            pltpu.sync_copy(
                rows_v.at[buf], out_hbm.at[pl.ds(base_b + g * NBG, NBG)]
            )

        fire(0, 0)
        n_outer = n_groups // 2

        def outer(t, carry):
            g0 = 2 * t
            fire(g0 + 1, 1)
            drain_and_store(g0, 0)

            @pl.when(t + 1 < n_outer)
            def _():
                fire(g0 + 2, 0)

            drain_and_store(g0 + 1, 1)
            return carry

        lax.fori_loop(0, n_outer, outer, 0)

    return body(idx, table)


def kernel(token_ids, embeddings):
    return _emb_lookup(token_ids.astype(jnp.int32), embeddings)


# speedup vs baseline: 2.2901x; 1.2179x over previous
import functools

import jax
import jax.numpy as jnp
from jax import lax
from jax.experimental import pallas as pl
from jax.experimental.pallas import tpu as pltpu
from jax.experimental.pallas import tpu_sc as plsc

DIM = 64
PDIM = 128
NC = 2
NS = 16
NW = NC * NS

NBG = 8


def _emb_lookup(idx, table):
    mesh = plsc.VectorSubcoreMesh(core_axis_name="c", subcore_axis_name="s")
    nbatch, seq = idx.shape
    b_per_w = nbatch // NW
    n_groups = b_per_w // NBG

    @functools.partial(
        pl.kernel,
        mesh=mesh,
        out_type=jax.ShapeDtypeStruct((nbatch, seq, PDIM), jnp.float32),
        scratch_types=[
            pltpu.VMEM((2, NBG, seq), jnp.int32),
            pltpu.VMEM((2, NBG, seq, PDIM), jnp.float32),
            pltpu.SemaphoreType.DMA,
            pltpu.SemaphoreType.DMA,
        ],
    )
    def body(idx_hbm, table_hbm, out_hbm, idx_v, rows_v, sem0, sem1):
        wid = lax.axis_index("s") * NC + lax.axis_index("c")
        base_b = wid * b_per_w
        sems = (sem0, sem1)

        def stage_and_fire(g, buf):
            pltpu.sync_copy(
                idx_hbm.at[pl.ds(base_b + g * NBG, NBG)], idx_v.at[buf]
            )
            sem = sems[buf]
            for i in range(NBG):
                pltpu.async_copy(
                    table_hbm.at[idx_v.at[buf, i]],
                    rows_v.at[buf, i],
                    sem,
                )

        def drain_and_store(g, buf):
            sem = sems[buf]
            for i in range(NBG):
                pltpu.make_async_copy(
                    table_hbm.at[idx_v.at[buf, i]], rows_v.at[buf, i], sem
                ).wait()
            pltpu.sync_copy(
                rows_v.at[buf], out_hbm.at[pl.ds(base_b + g * NBG, NBG)]
            )

        stage_and_fire(0, 0)
        n_outer = n_groups // 2

        def outer(t, carry):
            g0 = 2 * t
            stage_and_fire(g0 + 1, 1)
            drain_and_store(g0, 0)

            @pl.when(t + 1 < n_outer)
            def _():
                stage_and_fire(g0 + 2, 0)

            drain_and_store(g0 + 1, 1)
            return carry

        lax.fori_loop(0, n_outer, outer, 0)

    return body(idx, table)


def kernel(token_ids, embeddings):
    table_p = jnp.pad(embeddings, ((0, 0), (0, PDIM - DIM)))
    out_p = _emb_lookup(token_ids.astype(jnp.int32), table_p)
    return out_p[:, :, :DIM]


# speedup vs baseline: 2.8201x; 1.2314x over previous
import functools

import jax
import jax.numpy as jnp
from jax import lax
from jax.experimental import pallas as pl
from jax.experimental.pallas import tpu as pltpu
from jax.experimental.pallas import tpu_sc as plsc

DIM = 64
PDIM = 128
NC = 2
NS = 16
NW = NC * NS

NBG = 8


def _emb_lookup(idx, table):
    mesh = plsc.VectorSubcoreMesh(core_axis_name="c", subcore_axis_name="s")
    nbatch, seq = idx.shape
    b_per_w = nbatch // NW
    n_groups = b_per_w // NBG

    @functools.partial(
        pl.kernel,
        mesh=mesh,
        out_type=jax.ShapeDtypeStruct((nbatch, seq, PDIM), jnp.float32),
        scratch_types=[
            pltpu.VMEM((2, NBG, seq), jnp.int32),
            pltpu.VMEM((2, NBG, seq, PDIM), jnp.float32),
            pltpu.SemaphoreType.DMA,
            pltpu.SemaphoreType.DMA,
        ],
    )
    def body(idx_hbm, table_hbm, out_hbm, idx_v, rows_v, sem0, sem1):
        wid = lax.axis_index("s") * NC + lax.axis_index("c")
        base_b = wid * b_per_w
        sems = (sem0, sem1)

        def stage_and_fire(g, buf):
            pltpu.sync_copy(
                idx_hbm.at[pl.ds(base_b + g * NBG, NBG)], idx_v.at[buf]
            )
            sem = sems[buf]
            for i in range(NBG):
                pltpu.async_copy(
                    table_hbm.at[idx_v.at[buf, i]],
                    rows_v.at[buf, i],
                    sem,
                )

        def drain_and_store(g, buf):
            sem = sems[buf]
            for i in range(NBG):
                pltpu.make_async_copy(
                    table_hbm.at[idx_v.at[buf, i]], rows_v.at[buf, i], sem
                ).wait()
            pltpu.sync_copy(
                rows_v.at[buf], out_hbm.at[pl.ds(base_b + g * NBG, NBG)]
            )

        stage_and_fire(0, 0)
        n_outer = n_groups // 2

        def outer(t, carry):
            g0 = 2 * t
            stage_and_fire(g0 + 1, 1)
            drain_and_store(g0, 0)

            @pl.when(t + 1 < n_outer)
            def _():
                stage_and_fire(g0 + 2, 0)

            drain_and_store(g0 + 1, 1)
            return carry

        lax.fori_loop(0, n_outer, outer, 0)

    return body(idx, table)


TBLK = 4096


def _tpad_body(emb_t_ref, out_ref):
    out_ref[:, :DIM] = emb_t_ref[...].T


def _transpose_pad(emb_t):
    nrows = emb_t.shape[1]
    grid = (nrows + TBLK - 1) // TBLK
    return pl.pallas_call(
        _tpad_body,
        grid=(grid,),
        in_specs=[pl.BlockSpec((DIM, TBLK), lambda i: (0, i))],
        out_specs=pl.BlockSpec((TBLK, PDIM), lambda i: (i, 0)),
        out_shape=jax.ShapeDtypeStruct((nrows, PDIM), jnp.float32),
    )(emb_t)


def kernel(token_ids, embeddings):
    table_p = _transpose_pad(embeddings.T)
    out_p = _emb_lookup(token_ids.astype(jnp.int32), table_p)
    return out_p[:, :, :DIM]
